# Initial kernel scaffold; baseline (speedup 1.0000x reference)
import jax
import jax.numpy as jnp
from jax.experimental import pallas as pl


def kernel(lane_feats, edge_indexs, edge_attrs, params):
    raise NotImplementedError("write your pallas kernel here")


# speedup vs baseline: 2.7527x; 2.7527x over previous
import functools
import math

import jax
import jax.numpy as jnp
from jax import lax
from jax.experimental import pallas as pl
from jax.experimental.pallas import tpu as pltpu
from jax.experimental.pallas import tpu_sc as plsc

_N, _E, _D, _DE, _H, _L = 10000, 160000, 128, 16, 8, 3
_DH = _D // _H
_DF = 2 * _D
_SCALE = 1.0 / math.sqrt(_DH)

_NC, _NS = 2, 16
_NW = _NC * _NS
_CH = 128
_CHG = 200
_NCHUNKG = 25
_CHS = 400
_NCHUNKS = 25
_NCHUNK = 39
_EPW = _NCHUNK * _CH
_TAIL = _NW * _EPW
_ROWS = 624
_RTAIL = _NS * _ROWS
_NCHUNK1 = 78
_EPW1 = _NCHUNK1 * _CH
_TAIL1 = _NS * _EPW1

_BE = 2000
_BN = 2000

_f32 = jnp.float32


def _ln_tc(x, g, b):
    mu = jnp.mean(x, axis=-1, keepdims=True)
    var = jnp.mean((x - mu) ** 2, axis=-1, keepdims=True)
    return (x - mu) / jnp.sqrt(var + 1e-5) * g + b


def _dot(a, b):
    return jnp.dot(a, b, preferred_element_type=_f32)


def _head_sum_mask():
    r = lax.broadcasted_iota(jnp.int32, (_D, _H), 0) // _DH
    c = lax.broadcasted_iota(jnp.int32, (_D, _H), 1)
    return (r == c).astype(_f32)


def _head_expand_mask():
    r = lax.broadcasted_iota(jnp.int32, (_H, _D), 0)
    c = lax.broadcasted_iota(jnp.int32, (_H, _D), 1) // _DH
    return (r == c).astype(_f32)


def _head_embed16_mask():
    r = lax.broadcasted_iota(jnp.int32, (_H, 16), 0)
    c = lax.broadcasted_iota(jnp.int32, (_H, 16), 1)
    return (r == c).astype(_f32)


def _edge_core(ea, gtq, gs, wme, weu, wk, wv, aux, eao_ref, wvo_ref, w16_ref):
    gt = gtq[:, :_D]
    gq = gtq[:, _D:]
    ec = _dot(ea, wme)
    mem = jnp.maximum(_ln_tc(gt + gs + ec, aux[0:1], aux[1:2]), 0.0)
    delta = jnp.maximum(_ln_tc(_dot(mem, weu) + aux[2:3], aux[3:4], aux[4:5]), 0.0)
    eao_ref[...] = _ln_tc(ea + delta, aux[5:6], aux[6:7])
    kk = _dot(mem, wk)
    vv = _dot(mem, wv)
    logits = _dot(gq * kk, _head_sum_mask()) * _SCALE
    w = jnp.exp(logits)
    wex = _dot(w, _head_expand_mask())
    wvo_ref[...] = wex * vv
    w16_ref[...] = _dot(w, _head_embed16_mask())


def _edge_body_mid(ea_ref, gtq_ref, gs_ref, wme_ref, weu_ref, wk_ref, wv_ref,
                   aux_ref, eao_ref, wvo_ref, w16_ref):
    _edge_core(ea_ref[...], gtq_ref[...], gs_ref[...], wme_ref[...],
               weu_ref[...], wk_ref[...], wv_ref[...], aux_ref[...],
               eao_ref, wvo_ref, w16_ref)


def _edge_body_first(ear_ref, gtq_ref, gs_ref, wrpe_ref, auxr_ref,
                     wme_ref, weu_ref, wk_ref, wv_ref, aux_ref,
                     eao_ref, wvo_ref, w16_ref):
    auxr = auxr_ref[...]
    ea0 = jnp.maximum(
        _ln_tc(_dot(ear_ref[...], wrpe_ref[...]) + auxr[0:1], auxr[1:2], auxr[2:3]),
        0.0)
    _edge_core(ea0, gtq_ref[...], gs_ref[...], wme_ref[...],
               weu_ref[...], wk_ref[...], wv_ref[...], aux_ref[...],
               eao_ref, wvo_ref, w16_ref)


def _node_core(x, num, p2, wo, w1, b1, w2, aux):
    den8 = p2[:, 0:_H]
    dex = _dot(den8, _head_expand_mask())
    aggr = num / (dex + 1e-16)
    y = _dot(aggr, wo)
    x1 = _ln_tc(x + y, aux[1:2], aux[2:3])
    h = jnp.maximum(_dot(x1, w1) + b1[0:1], 0.0)
    h2 = _dot(h, w2) + aux[0:1]
    return _ln_tc(x1 + h2, aux[3:4], aux[4:5])


def _node_body_mid(x_ref, p1_ref, p2_ref, wo_ref, w1_ref, b1_ref, w2_ref,
                   aux_ref, wmt_ref, wms_ref, wq_ref,
                   xo_ref, tq_ref, s_ref):
    aux = aux_ref[...]
    x2 = _node_core(x_ref[...], p1_ref[...], p2_ref[...],
                    wo_ref[...], w1_ref[...], b1_ref[...], w2_ref[...], aux)
    xo_ref[...] = x2
    tq_ref[...] = jnp.concatenate(
        [_dot(x2, wmt_ref[...]) + aux[5:6], _dot(x2, wq_ref[...])], axis=1)
    s_ref[...] = _dot(x2, wms_ref[...])


def _node_body_final(x_ref, p1_ref, p2_ref, wo_ref, w1_ref, b1_ref, w2_ref,
                     aux_ref, xo_ref):
    xo_ref[...] = _node_core(x_ref[...], p1_ref[...], p2_ref[...],
                             wo_ref[...], w1_ref[...], b1_ref[...], w2_ref[...],
                             aux_ref[...])


def _idxprep_body(d_ref, l0_ref, l1_ref):
    d = d_ref[0]
    l0_ref[0] = jnp.where(d < _NH, d, _NH)
    l1_ref[0] = jnp.where(d >= _NH, d - _NH, _NH)


def _idxprep_call(dst):
    nb = _E // _BE
    d3 = dst.reshape(nb, 1, _BE)
    l0, l1 = pl.pallas_call(
        _idxprep_body,
        grid=(nb,),
        in_specs=[pl.BlockSpec((1, 1, _BE), lambda i: (i, 0, 0))],
        out_specs=(pl.BlockSpec((1, 1, _BE), lambda i: (i, 0, 0)),
                   pl.BlockSpec((1, 1, _BE), lambda i: (i, 0, 0))),
        out_shape=(jax.ShapeDtypeStruct((nb, 1, _BE), jnp.int32),
                   jax.ShapeDtypeStruct((nb, 1, _BE), jnp.int32)),
    )(d3)
    return jnp.stack([l0.reshape(_E), l1.reshape(_E)])


def _node_body_prep(x_ref, wmt_ref, wms_ref, wq_ref, aux_ref, tq_ref, s_ref):
    x = x_ref[...]
    aux = aux_ref[...]
    tq_ref[...] = jnp.concatenate(
        [_dot(x, wmt_ref[...]) + aux[0:1], _dot(x, wq_ref[...])], axis=1)
    s_ref[...] = _dot(x, wms_ref[...])


def _full(*shape):
    return pl.BlockSpec(shape, lambda i: tuple(0 for _ in shape))


def _rows(bs, w):
    return pl.BlockSpec((bs, w), lambda i: (i, 0))


def _edge_call(ea, gtq, gs, wme, weu, wk, wv, aux, first_args=None):
    grid = (_E // _BE,)
    out_shape = (
        jax.ShapeDtypeStruct((_E, _D), _f32),
        jax.ShapeDtypeStruct((_E, _D), _f32),
        jax.ShapeDtypeStruct((_E, 16), _f32),
    )
    out_specs = (_rows(_BE, _D), _rows(_BE, _D), _rows(_BE, 16))
    if first_args is None:
        return pl.pallas_call(
            _edge_body_mid,
            grid=grid,
            in_specs=[_rows(_BE, _D), _rows(_BE, 2 * _D), _rows(_BE, _D),
                      _full(_D, _D), _full(_D, _D), _full(_D, _D), _full(_D, _D),
                      _full(8, _D)],
            out_specs=out_specs,
            out_shape=out_shape,
        )(ea, gtq, gs, wme, weu, wk, wv, aux)
    wrpe, auxr = first_args
    return pl.pallas_call(
        _edge_body_first,
        grid=grid,
        in_specs=[_rows(_BE, _DE), _rows(_BE, 2 * _D), _rows(_BE, _D),
                  _full(_DE, _D), _full(8, _D),
                  _full(_D, _D), _full(_D, _D), _full(_D, _D), _full(_D, _D),
                  _full(8, _D)],
        out_specs=out_specs,
        out_shape=out_shape,
    )(ea, gtq, gs, wrpe, auxr, wme, weu, wk, wv, aux)


def _node_call_mid(x, p1, p2, wo, w1, b1, w2, aux, wmt, wms, wq):
    grid = (_N // _BN,)
    return pl.pallas_call(
        _node_body_mid,
        grid=grid,
        in_specs=[_rows(_BN, _D), _rows(_BN, _D), _rows(_BN, 16),
                  _full(_D, _D), _full(_D, _DF), _full(8, _DF), _full(_DF, _D),
                  _full(8, _D), _full(_D, _D), _full(_D, _D), _full(_D, _D)],
        out_specs=(_rows(_BN, _D), _rows(_BN, 2 * _D), _rows(_BN, _D)),
        out_shape=(jax.ShapeDtypeStruct((_N, _D), _f32),
                   jax.ShapeDtypeStruct((_N, 2 * _D), _f32),
                   jax.ShapeDtypeStruct((_N, _D), _f32)),
    )(x, p1, p2, wo, w1, b1, w2, aux, wmt, wms, wq)


def _node_call_final(x, p1, p2, wo, w1, b1, w2, aux):
    grid = (_N // _BN,)
    return pl.pallas_call(
        _node_body_final,
        grid=grid,
        in_specs=[_rows(_BN, _D), _rows(_BN, _D), _rows(_BN, 16),
                  _full(_D, _D), _full(_D, _DF), _full(8, _DF), _full(_DF, _D),
                  _full(8, _D)],
        out_specs=_rows(_BN, _D),
        out_shape=jax.ShapeDtypeStruct((_N, _D), _f32),
    )(x, p1, p2, wo, w1, b1, w2, aux)


def _node_call_prep(x, wmt, wms, wq, aux):
    grid = (_N // _BN,)
    return pl.pallas_call(
        _node_body_prep,
        grid=grid,
        in_specs=[_rows(_BN, _D), _full(_D, _D), _full(_D, _D), _full(_D, _D),
                  _full(8, _D)],
        out_specs=(_rows(_BN, 2 * _D), _rows(_BN, _D)),
        out_shape=(jax.ShapeDtypeStruct((_N, 2 * _D), _f32),
                   jax.ShapeDtypeStruct((_N, _D), _f32)),
    )(x, wmt, wms, wq, aux)


def _sc_mesh():
    return plsc.VectorSubcoreMesh(core_axis_name="c", subcore_axis_name="s")


def _gather_call(tq_tab, s_tab, dst, src):
    @functools.partial(
        pl.kernel,
        out_type=(jax.ShapeDtypeStruct((_E, 2 * _D), _f32),
                  jax.ShapeDtypeStruct((_E, _D), _f32)),
        mesh=_sc_mesh(),
        compiler_params=pltpu.CompilerParams(use_tc_tiling_on_sc=False),
        scratch_types=[
            pltpu.VMEM((_CHG,), jnp.int32),
            pltpu.VMEM((_CHG,), jnp.int32),
            pltpu.VMEM((_CHG, 2 * _D), _f32),
            pltpu.VMEM((_CHG, _D), _f32),
            pltpu.SemaphoreType.DMA,
            pltpu.SemaphoreType.DMA,
        ],
    )
    def gk(tq_hbm, s_hbm, dst_hbm, src_hbm, gtq_hbm, gs_hbm,
           idx_d, idx_s, rows_tq, rows_s, sem1, sem2):
        wid = lax.axis_index("s") * _NC + lax.axis_index("c")

        def do(off):
            pltpu.sync_copy(dst_hbm.at[pl.ds(off, _CHG)], idx_d)
            pltpu.sync_copy(src_hbm.at[pl.ds(off, _CHG)], idx_s)
            cp1 = pltpu.async_copy(tq_hbm.at[idx_d], rows_tq, sem1)
            cp2 = pltpu.async_copy(s_hbm.at[idx_s], rows_s, sem2)
            cp1.wait()
            cp2.wait()
            pltpu.sync_copy(rows_tq, gtq_hbm.at[pl.ds(off, _CHG)])
            pltpu.sync_copy(rows_s, gs_hbm.at[pl.ds(off, _CHG)])

        def body(j, carry):
            do(wid * (_E // _NW) + j * _CHG)
            return carry

        lax.fori_loop(0, _NCHUNKG, body, 0)

    return gk(tq_tab, s_tab, dst, src)


_NH = _N // 2
_ACCR = _NH + 8
_WROWS = 312
_WTAIL = _NS * _WROWS


def _scatter_half(wv, w16, dstl, z1, z2):
    @functools.partial(
        pl.kernel,
        out_type=(jax.ShapeDtypeStruct((_ACCR, _D), _f32),
                  jax.ShapeDtypeStruct((_ACCR, 16), _f32)),
        mesh=_sc_mesh(),
        compiler_params=pltpu.CompilerParams(use_tc_tiling_on_sc=False),
        scratch_types=[
            pltpu.VMEM((_CHS,), jnp.int32),
            pltpu.VMEM((_CHS, _D), _f32),
            pltpu.VMEM((_CHS, 16), _f32),
            pltpu.VMEM((104, _D), _f32),
            pltpu.VMEM((104, 16), _f32),
            pltpu.VMEM_SHARED((_ACCR, _D), _f32),
            pltpu.VMEM_SHARED((_ACCR, 16), _f32),
        ],
    )
    def sk(wv_hbm, w16_hbm, dstl_hbm, z1_hbm, z2_hbm, p1_hbm, p2_hbm,
           idx_v, pay1_v, pay2_v, zb1, zb2, acc1, acc2):
        cidx = lax.axis_index("c")
        sidx = lax.axis_index("s")
        base = sidx * _WROWS

        @pl.when(cidx == 0)
        def _core0_work():
            pltpu.sync_copy(z1_hbm, zb1)
            pltpu.sync_copy(z2_hbm, zb2)

            def zbody(r, carry):
                off = base + r * 104
                pltpu.sync_copy(zb1, acc1.at[pl.ds(off, 104)])
                pltpu.sync_copy(zb2, acc2.at[pl.ds(off, 104)])
                return carry

            lax.fori_loop(0, _WROWS // 104, zbody, 0)

            @pl.when(sidx < 2)
            def _():
                off = _WTAIL + sidx * 8
                pltpu.sync_copy(zb1.at[pl.ds(0, 8)], acc1.at[pl.ds(off, 8)])
                pltpu.sync_copy(zb2.at[pl.ds(0, 8)], acc2.at[pl.ds(off, 8)])

            plsc.subcore_barrier()

            def do(off):
                pltpu.sync_copy(dstl_hbm.at[pl.ds(off, _CHS)], idx_v)
                pltpu.sync_copy(wv_hbm.at[pl.ds(off, _CHS)], pay1_v)
                pltpu.sync_copy(w16_hbm.at[pl.ds(off, _CHS)], pay2_v)
                pltpu.sync_copy(pay1_v, acc1.at[idx_v], add=True)
                pltpu.sync_copy(pay2_v, acc2.at[idx_v], add=True)

            def body(jj, carry):
                do(sidx * (_E // _NS) + jj * _CHS)
                return carry

            lax.fori_loop(0, _NCHUNKS, body, 0)

            plsc.subcore_barrier()

            def wbody(r, carry):
                off = base + r * 104
                pltpu.sync_copy(acc1.at[pl.ds(off, 104)], zb1)
                pltpu.sync_copy(acc2.at[pl.ds(off, 104)], zb2)
                pltpu.sync_copy(zb1, p1_hbm.at[pl.ds(off, 104)])
                pltpu.sync_copy(zb2, p2_hbm.at[pl.ds(off, 104)])
                return carry

            lax.fori_loop(0, _WROWS // 104, wbody, 0)

            @pl.when(sidx < 2)
            def _():
                off = _WTAIL + sidx * 8
                pltpu.sync_copy(acc1.at[pl.ds(off, 8)], zb1.at[pl.ds(0, 8)])
                pltpu.sync_copy(acc2.at[pl.ds(off, 8)], zb2.at[pl.ds(0, 8)])
                pltpu.sync_copy(zb1.at[pl.ds(0, 8)], p1_hbm.at[pl.ds(off, 8)])
                pltpu.sync_copy(zb2.at[pl.ds(0, 8)], p2_hbm.at[pl.ds(off, 8)])

    return sk(wv, w16, dstl, z1, z2)


def _scatter_call(wv, w16, dstl, z1, z2):
    p1a, p2a = _scatter_half(wv, w16, dstl[0], z1, z2)
    p1b, p2b = _scatter_half(wv, w16, dstl[1], z1, z2)
    return (jnp.concatenate([p1a[:_NH], p1b[:_NH]], axis=0),
            jnp.concatenate([p2a[:_NH], p2b[:_NH]], axis=0))


def _edge_aux(p):
    return jnp.stack([p['ln_mem_g'], p['ln_mem_b'], p['b_eu'], p['ln_eu_g'],
                      p['ln_eu_b'], p['ln_e_g'], p['ln_e_b'],
                      jnp.zeros((_D,), _f32)])


def _node_aux(p, b_mem_next):
    return jnp.stack([p['b2'], p['ln1_g'], p['ln1_b'], p['ln2_g'], p['ln2_b'],
                      b_mem_next, jnp.zeros((_D,), _f32),
                      jnp.zeros((_D,), _f32)])


def kernel(lane_feats, edge_indexs, edge_attrs, params):
    src = edge_indexs[0]
    dst = edge_indexs[1]
    layers = params['layers']

    z1 = jnp.zeros((104, _D), _f32)
    z2 = jnp.zeros((104, 16), _f32)
    zrow = jnp.zeros((_D,), _f32)

    auxr = jnp.stack([params['b_rpe'], params['ln_rpe_g'], params['ln_rpe_b'],
                      zrow, zrow, zrow, zrow, zrow])

    def wsplit(p):
        wm = p['W_mem']
        return wm[0:_D], wm[_D:2 * _D], wm[2 * _D:3 * _D]

    x = lane_feats
    ea = edge_attrs
    dstl = _idxprep_call(dst)

    wmt0, wms0, wme0 = wsplit(layers[0])
    prep_aux = jnp.stack([layers[0]['b_mem']] + [zrow] * 7)
    tq, s_tab = _node_call_prep(x, wmt0, wms0, layers[0]['Wq'], prep_aux)

    for li in range(_L):
        p = layers[li]
        _, _, wme = wsplit(p)
        gtq, gs = _gather_call(tq, s_tab, dst, src)
        first_args = (params['W_rpe'], auxr) if li == 0 else None
        ea, wv, w16 = _edge_call(ea, gtq, gs, wme, p['W_eu'], p['Wk'], p['Wv'],
                                 _edge_aux(p), first_args=first_args)
        p1, p2 = _scatter_call(wv, w16, dstl, z1, z2)
        b1 = jnp.zeros((8, _DF), _f32).at[0].set(p['b1'])
        if li + 1 < _L:
            pn = layers[li + 1]
            wmt_n, wms_n, _ = wsplit(pn)
            x, tq, s_tab = _node_call_mid(
                x, p1, p2, p['Wo'], p['W1'], b1, p['W2'],
                _node_aux(p, pn['b_mem']), wmt_n, wms_n, pn['Wq'])
        else:
            x = _node_call_final(x, p1, p2, p['Wo'], p['W1'], b1, p['W2'],
                                 _node_aux(p, zrow))
    return x


# speedup vs baseline: 3.1941x; 1.1603x over previous
import functools
import math

import jax
import jax.numpy as jnp
from jax import lax
from jax.experimental import pallas as pl
from jax.experimental.pallas import tpu as pltpu
from jax.experimental.pallas import tpu_sc as plsc

_N, _E, _D, _DE, _H, _L = 10000, 160000, 128, 16, 8, 3
_DH = _D // _H
_DF = 2 * _D
_SCALE = 1.0 / math.sqrt(_DH)

_NC, _NS = 2, 16
_NW = _NC * _NS
_CH = 128
_CHG = 200
_NCHUNKG = 25
_CHS = 400
_NCHUNKS = 25
_NCHUNK = 39
_EPW = _NCHUNK * _CH
_TAIL = _NW * _EPW
_ROWS = 624
_RTAIL = _NS * _ROWS
_NCHUNK1 = 78
_EPW1 = _NCHUNK1 * _CH
_TAIL1 = _NS * _EPW1

_BE = 2000
_BN = 2000

_f32 = jnp.float32


def _ln_tc(x, g, b):
    mu = jnp.mean(x, axis=-1, keepdims=True)
    var = jnp.mean((x - mu) ** 2, axis=-1, keepdims=True)
    return (x - mu) / jnp.sqrt(var + 1e-5) * g + b


def _dot(a, b):
    return jnp.dot(a, b, preferred_element_type=_f32)


def _head_sum_mask():
    r = lax.broadcasted_iota(jnp.int32, (_D, _H), 0) // _DH
    c = lax.broadcasted_iota(jnp.int32, (_D, _H), 1)
    return (r == c).astype(_f32)


def _head_expand_mask():
    r = lax.broadcasted_iota(jnp.int32, (_H, _D), 0)
    c = lax.broadcasted_iota(jnp.int32, (_H, _D), 1) // _DH
    return (r == c).astype(_f32)


def _head_embed16_mask():
    r = lax.broadcasted_iota(jnp.int32, (_H, 16), 0)
    c = lax.broadcasted_iota(jnp.int32, (_H, 16), 1)
    return (r == c).astype(_f32)


def _edge_core(ea, gtq, gs, wme, weu, wk, wv, aux, eao_ref, wvo_ref, w16_ref):
    gt = gtq[:, :_D]
    gq = gtq[:, _D:]
    ec = _dot(ea, wme)
    mem = jnp.maximum(_ln_tc(gt + gs + ec, aux[0:1], aux[1:2]), 0.0)
    delta = jnp.maximum(_ln_tc(_dot(mem, weu) + aux[2:3], aux[3:4], aux[4:5]), 0.0)
    eao_ref[...] = _ln_tc(ea + delta, aux[5:6], aux[6:7])
    kk = _dot(mem, wk)
    vv = _dot(mem, wv)
    logits = _dot(gq * kk, _head_sum_mask()) * _SCALE
    w = jnp.exp(logits)
    wex = _dot(w, _head_expand_mask())
    wvo_ref[...] = wex * vv
    w16_ref[...] = _dot(w, _head_embed16_mask())


def _edge_body_mid(ea_ref, gtq_ref, gs_ref, wme_ref, weu_ref, wk_ref, wv_ref,
                   aux_ref, eao_ref, wvo_ref, w16_ref):
    _edge_core(ea_ref[...], gtq_ref[...], gs_ref[...], wme_ref[...],
               weu_ref[...], wk_ref[...], wv_ref[...], aux_ref[...],
               eao_ref, wvo_ref, w16_ref)


def _edge_body_first(ear_ref, gtq_ref, gs_ref, wrpe_ref, auxr_ref,
                     wme_ref, weu_ref, wk_ref, wv_ref, aux_ref,
                     eao_ref, wvo_ref, w16_ref):
    auxr = auxr_ref[...]
    ea0 = jnp.maximum(
        _ln_tc(_dot(ear_ref[...], wrpe_ref[...]) + auxr[0:1], auxr[1:2], auxr[2:3]),
        0.0)
    _edge_core(ea0, gtq_ref[...], gs_ref[...], wme_ref[...],
               weu_ref[...], wk_ref[...], wv_ref[...], aux_ref[...],
               eao_ref, wvo_ref, w16_ref)


def _node_core(x, num, p2, wo, w1, b1, w2, aux):
    den8 = p2[:, 0:_H]
    dex = _dot(den8, _head_expand_mask())
    aggr = num / (dex + 1e-16)
    y = _dot(aggr, wo)
    x1 = _ln_tc(x + y, aux[1:2], aux[2:3])
    h = jnp.maximum(_dot(x1, w1) + b1[0:1], 0.0)
    h2 = _dot(h, w2) + aux[0:1]
    return _ln_tc(x1 + h2, aux[3:4], aux[4:5])


def _node_body_mid(x_ref, p1_ref, p2_ref, wo_ref, w1_ref, b1_ref, w2_ref,
                   aux_ref, wmt_ref, wms_ref, wq_ref,
                   xo_ref, tq_ref, s_ref):
    aux = aux_ref[...]
    x2 = _node_core(x_ref[...], p1_ref[...], p2_ref[...],
                    wo_ref[...], w1_ref[...], b1_ref[...], w2_ref[...], aux)
    xo_ref[...] = x2
    tq_ref[...] = jnp.concatenate(
        [_dot(x2, wmt_ref[...]) + aux[5:6], _dot(x2, wq_ref[...])], axis=1)
    s_ref[...] = _dot(x2, wms_ref[...])


def _node_body_final(x_ref, p1_ref, p2_ref, wo_ref, w1_ref, b1_ref, w2_ref,
                     aux_ref, xo_ref):
    xo_ref[...] = _node_core(x_ref[...], p1_ref[...], p2_ref[...],
                             wo_ref[...], w1_ref[...], b1_ref[...], w2_ref[...],
                             aux_ref[...])


def _idxprep_body(d_ref, l0_ref, l1_ref):
    d = d_ref[0]
    l0_ref[0] = jnp.where(d < _NH, d, _NH)
    l1_ref[0] = jnp.where(d >= _NH, d - _NH, _NH)


def _idxprep_call(dst):
    nb = _E // _BE
    d3 = dst.reshape(nb, 1, _BE)
    l0, l1 = pl.pallas_call(
        _idxprep_body,
        grid=(nb,),
        in_specs=[pl.BlockSpec((1, 1, _BE), lambda i: (i, 0, 0))],
        out_specs=(pl.BlockSpec((1, 1, _BE), lambda i: (i, 0, 0)),
                   pl.BlockSpec((1, 1, _BE), lambda i: (i, 0, 0))),
        out_shape=(jax.ShapeDtypeStruct((nb, 1, _BE), jnp.int32),
                   jax.ShapeDtypeStruct((nb, 1, _BE), jnp.int32)),
    )(d3)
    return jnp.stack([l0.reshape(_E), l1.reshape(_E)])


def _node_body_prep(x_ref, wmt_ref, wms_ref, wq_ref, aux_ref, tq_ref, s_ref):
    x = x_ref[...]
    aux = aux_ref[...]
    tq_ref[...] = jnp.concatenate(
        [_dot(x, wmt_ref[...]) + aux[0:1], _dot(x, wq_ref[...])], axis=1)
    s_ref[...] = _dot(x, wms_ref[...])


def _full(*shape):
    return pl.BlockSpec(shape, lambda i: tuple(0 for _ in shape))


def _rows(bs, w):
    return pl.BlockSpec((bs, w), lambda i: (i, 0))


def _edge_call(ea, gtq, gs, wme, weu, wk, wv, aux, first_args=None):
    grid = (_E // _BE,)
    out_shape = (
        jax.ShapeDtypeStruct((_E, _D), _f32),
        jax.ShapeDtypeStruct((_E, _D), _f32),
        jax.ShapeDtypeStruct((_E, 16), _f32),
    )
    out_specs = (_rows(_BE, _D), _rows(_BE, _D), _rows(_BE, 16))
    if first_args is None:
        return pl.pallas_call(
            _edge_body_mid,
            grid=grid,
            in_specs=[_rows(_BE, _D), _rows(_BE, 2 * _D), _rows(_BE, _D),
                      _full(_D, _D), _full(_D, _D), _full(_D, _D), _full(_D, _D),
                      _full(8, _D)],
            out_specs=out_specs,
            out_shape=out_shape,
        )(ea, gtq, gs, wme, weu, wk, wv, aux)
    wrpe, auxr = first_args
    return pl.pallas_call(
        _edge_body_first,
        grid=grid,
        in_specs=[_rows(_BE, _DE), _rows(_BE, 2 * _D), _rows(_BE, _D),
                  _full(_DE, _D), _full(8, _D),
                  _full(_D, _D), _full(_D, _D), _full(_D, _D), _full(_D, _D),
                  _full(8, _D)],
        out_specs=out_specs,
        out_shape=out_shape,
    )(ea, gtq, gs, wrpe, auxr, wme, weu, wk, wv, aux)


def _node_call_mid(x, p1, p2, wo, w1, b1, w2, aux, wmt, wms, wq):
    grid = (_N // _BN,)
    return pl.pallas_call(
        _node_body_mid,
        grid=grid,
        in_specs=[_rows(_BN, _D), _rows(_BN, _D), _rows(_BN, 16),
                  _full(_D, _D), _full(_D, _DF), _full(8, _DF), _full(_DF, _D),
                  _full(8, _D), _full(_D, _D), _full(_D, _D), _full(_D, _D)],
        out_specs=(_rows(_BN, _D), _rows(_BN, 2 * _D), _rows(_BN, _D)),
        out_shape=(jax.ShapeDtypeStruct((_N, _D), _f32),
                   jax.ShapeDtypeStruct((_N, 2 * _D), _f32),
                   jax.ShapeDtypeStruct((_N, _D), _f32)),
    )(x, p1, p2, wo, w1, b1, w2, aux, wmt, wms, wq)


def _node_call_final(x, p1, p2, wo, w1, b1, w2, aux):
    grid = (_N // _BN,)
    return pl.pallas_call(
        _node_body_final,
        grid=grid,
        in_specs=[_rows(_BN, _D), _rows(_BN, _D), _rows(_BN, 16),
                  _full(_D, _D), _full(_D, _DF), _full(8, _DF), _full(_DF, _D),
                  _full(8, _D)],
        out_specs=_rows(_BN, _D),
        out_shape=jax.ShapeDtypeStruct((_N, _D), _f32),
    )(x, p1, p2, wo, w1, b1, w2, aux)


def _node_call_prep(x, wmt, wms, wq, aux):
    grid = (_N // _BN,)
    return pl.pallas_call(
        _node_body_prep,
        grid=grid,
        in_specs=[_rows(_BN, _D), _full(_D, _D), _full(_D, _D), _full(_D, _D),
                  _full(8, _D)],
        out_specs=(_rows(_BN, 2 * _D), _rows(_BN, _D)),
        out_shape=(jax.ShapeDtypeStruct((_N, 2 * _D), _f32),
                   jax.ShapeDtypeStruct((_N, _D), _f32)),
    )(x, wmt, wms, wq, aux)


def _sc_mesh():
    return plsc.VectorSubcoreMesh(core_axis_name="c", subcore_axis_name="s")


def _gather_call(tq_tab, s_tab, dst, src):
    @functools.partial(
        pl.kernel,
        out_type=(jax.ShapeDtypeStruct((_E, 2 * _D), _f32),
                  jax.ShapeDtypeStruct((_E, _D), _f32)),
        mesh=_sc_mesh(),
        scratch_types=[
            pltpu.VMEM((_CH,), jnp.int32),
            pltpu.VMEM((_CH,), jnp.int32),
            pltpu.VMEM((_CH, 2 * _D), _f32),
            pltpu.VMEM((_CH, _D), _f32),
            pltpu.SemaphoreType.DMA,
            pltpu.SemaphoreType.DMA,
        ],
    )
    def gk(tq_hbm, s_hbm, dst_hbm, src_hbm, gtq_hbm, gs_hbm,
           idx_d, idx_s, rows_tq, rows_s, sem1, sem2):
        wid = lax.axis_index("s") * _NC + lax.axis_index("c")

        def do(off):
            pltpu.sync_copy(dst_hbm.at[pl.ds(off, _CH)], idx_d)
            pltpu.sync_copy(src_hbm.at[pl.ds(off, _CH)], idx_s)
            cp1 = pltpu.async_copy(tq_hbm.at[idx_d], rows_tq, sem1)
            cp2 = pltpu.async_copy(s_hbm.at[idx_s], rows_s, sem2)
            cp1.wait()
            cp2.wait()
            pltpu.sync_copy(rows_tq, gtq_hbm.at[pl.ds(off, _CH)])
            pltpu.sync_copy(rows_s, gs_hbm.at[pl.ds(off, _CH)])

        def body(j, carry):
            do(wid * _EPW + j * _CH)
            return carry

        lax.fori_loop(0, _NCHUNK, body, 0)

        @pl.when(wid < 2)
        def _():
            do(_TAIL + wid * _CH)

    return gk(tq_tab, s_tab, dst, src)


_NH = _N // 2
_ACCR = _NH + 8
_WROWS = 312
_WTAIL = _NS * _WROWS


def _scatter_half(wv, w16, dstl, z1, z2):
    @functools.partial(
        pl.kernel,
        out_type=(jax.ShapeDtypeStruct((_ACCR, _D), _f32),
                  jax.ShapeDtypeStruct((_ACCR, 16), _f32)),
        mesh=_sc_mesh(),
        compiler_params=pltpu.CompilerParams(use_tc_tiling_on_sc=False),
        scratch_types=[
            pltpu.VMEM((_CHS,), jnp.int32),
            pltpu.VMEM((_CHS, _D), _f32),
            pltpu.VMEM((_CHS, 16), _f32),
            pltpu.VMEM((104, _D), _f32),
            pltpu.VMEM((104, 16), _f32),
            pltpu.VMEM_SHARED((_ACCR, _D), _f32),
            pltpu.VMEM_SHARED((_ACCR, 16), _f32),
        ],
    )
    def sk(wv_hbm, w16_hbm, dstl_hbm, z1_hbm, z2_hbm, p1_hbm, p2_hbm,
           idx_v, pay1_v, pay2_v, zb1, zb2, acc1, acc2):
        cidx = lax.axis_index("c")
        sidx = lax.axis_index("s")
        base = sidx * _WROWS

        @pl.when(cidx == 0)
        def _core0_work():
            pltpu.sync_copy(z1_hbm, zb1)
            pltpu.sync_copy(z2_hbm, zb2)

            def zbody(r, carry):
                off = base + r * 104
                pltpu.sync_copy(zb1, acc1.at[pl.ds(off, 104)])
                pltpu.sync_copy(zb2, acc2.at[pl.ds(off, 104)])
                return carry

            lax.fori_loop(0, _WROWS // 104, zbody, 0)

            @pl.when(sidx < 2)
            def _():
                off = _WTAIL + sidx * 8
                pltpu.sync_copy(zb1.at[pl.ds(0, 8)], acc1.at[pl.ds(off, 8)])
                pltpu.sync_copy(zb2.at[pl.ds(0, 8)], acc2.at[pl.ds(off, 8)])

            plsc.subcore_barrier()

            def do(off):
                pltpu.sync_copy(dstl_hbm.at[pl.ds(off, _CHS)], idx_v)
                pltpu.sync_copy(wv_hbm.at[pl.ds(off, _CHS)], pay1_v)
                pltpu.sync_copy(w16_hbm.at[pl.ds(off, _CHS)], pay2_v)
                pltpu.sync_copy(pay1_v, acc1.at[idx_v], add=True)
                pltpu.sync_copy(pay2_v, acc2.at[idx_v], add=True)

            def body(jj, carry):
                do(sidx * (_E // _NS) + jj * _CHS)
                return carry

            lax.fori_loop(0, _NCHUNKS, body, 0)

            plsc.subcore_barrier()

            def wbody(r, carry):
                off = base + r * 104
                pltpu.sync_copy(acc1.at[pl.ds(off, 104)], zb1)
                pltpu.sync_copy(acc2.at[pl.ds(off, 104)], zb2)
                pltpu.sync_copy(zb1, p1_hbm.at[pl.ds(off, 104)])
                pltpu.sync_copy(zb2, p2_hbm.at[pl.ds(off, 104)])
                return carry

            lax.fori_loop(0, _WROWS // 104, wbody, 0)

            @pl.when(sidx < 2)
            def _():
                off = _WTAIL + sidx * 8
                pltpu.sync_copy(acc1.at[pl.ds(off, 8)], zb1.at[pl.ds(0, 8)])
                pltpu.sync_copy(acc2.at[pl.ds(off, 8)], zb2.at[pl.ds(0, 8)])
                pltpu.sync_copy(zb1.at[pl.ds(0, 8)], p1_hbm.at[pl.ds(off, 8)])
                pltpu.sync_copy(zb2.at[pl.ds(0, 8)], p2_hbm.at[pl.ds(off, 8)])

    return sk(wv, w16, dstl, z1, z2)


def _scatter_call(wv, w16, dstl, z1, z2):
    p1a, p2a = _scatter_half(wv, w16, dstl[0], z1, z2)
    p1b, p2b = _scatter_half(wv, w16, dstl[1], z1, z2)
    return (jnp.concatenate([p1a[:_NH], p1b[:_NH]], axis=0),
            jnp.concatenate([p2a[:_NH], p2b[:_NH]], axis=0))


def _edge_aux(p):
    return jnp.stack([p['ln_mem_g'], p['ln_mem_b'], p['b_eu'], p['ln_eu_g'],
                      p['ln_eu_b'], p['ln_e_g'], p['ln_e_b'],
                      jnp.zeros((_D,), _f32)])


def _node_aux(p, b_mem_next):
    return jnp.stack([p['b2'], p['ln1_g'], p['ln1_b'], p['ln2_g'], p['ln2_b'],
                      b_mem_next, jnp.zeros((_D,), _f32),
                      jnp.zeros((_D,), _f32)])


def kernel(lane_feats, edge_indexs, edge_attrs, params):
    src = edge_indexs[0]
    dst = edge_indexs[1]
    layers = params['layers']

    z1 = jnp.zeros((104, _D), _f32)
    z2 = jnp.zeros((104, 16), _f32)
    zrow = jnp.zeros((_D,), _f32)

    auxr = jnp.stack([params['b_rpe'], params['ln_rpe_g'], params['ln_rpe_b'],
                      zrow, zrow, zrow, zrow, zrow])

    def wsplit(p):
        wm = p['W_mem']
        return wm[0:_D], wm[_D:2 * _D], wm[2 * _D:3 * _D]

    x = lane_feats
    ea = edge_attrs
    dstl = _idxprep_call(dst)

    wmt0, wms0, wme0 = wsplit(layers[0])
    prep_aux = jnp.stack([layers[0]['b_mem']] + [zrow] * 7)
    tq, s_tab = _node_call_prep(x, wmt0, wms0, layers[0]['Wq'], prep_aux)

    for li in range(_L):
        p = layers[li]
        _, _, wme = wsplit(p)
        gtq, gs = _gather_call(tq, s_tab, dst, src)
        first_args = (params['W_rpe'], auxr) if li == 0 else None
        ea, wv, w16 = _edge_call(ea, gtq, gs, wme, p['W_eu'], p['Wk'], p['Wv'],
                                 _edge_aux(p), first_args=first_args)
        p1, p2 = _scatter_call(wv, w16, dstl, z1, z2)
        b1 = jnp.zeros((8, _DF), _f32).at[0].set(p['b1'])
        if li + 1 < _L:
            pn = layers[li + 1]
            wmt_n, wms_n, _ = wsplit(pn)
            x, tq, s_tab = _node_call_mid(
                x, p1, p2, p['Wo'], p['W1'], b1, p['W2'],
                _node_aux(p, pn['b_mem']), wmt_n, wms_n, pn['Wq'])
        else:
            x = _node_call_final(x, p1, p2, p['Wo'], p['W1'], b1, p['W2'],
                                 _node_aux(p, zrow))
    return x
